# speedup vs baseline: 89.3574x; 89.3574x over previous
import functools

import jax
import jax.numpy as jnp
from jax import lax
from jax.experimental import pallas as pl
from jax.experimental.pallas import tpu as pltpu
from jax.experimental.pallas import tpu_sc as plsc

_N = 50000
_S = 1600000
_T = 512
_R = 5
_NW = 32
_SYN_W = _S // _NW
_CHUNK = 2000
_NCHUNK = _SYN_W // _CHUNK
_VREGS = _CHUNK // 16
_ACC = 51200

_mesh = plsc.VectorSubcoreMesh(core_axis_name="c", subcore_axis_name="s")
_cp = pltpu.CompilerParams(needs_layout_passes=False)


@functools.partial(
    pl.kernel,
    out_type=jax.ShapeDtypeStruct((_NW, _ACC), jnp.float32),
    mesh=_mesh,
    compiler_params=_cp,
    scratch_types=[
        pltpu.VMEM((_N,), jnp.float32),
        pltpu.VMEM((_T * _R,), jnp.float32),
        pltpu.VMEM((_T,), jnp.float32),
        pltpu.VMEM((_ACC,), jnp.float32),
        pltpu.VMEM((_CHUNK,), jnp.int32),
        pltpu.VMEM((_CHUNK,), jnp.int32),
        pltpu.VMEM((_CHUNK,), jnp.int32),
        pltpu.VMEM((_CHUNK,), jnp.float32),
    ],
)
def _sc_synapse_kernel(pre_hbm, post_hbm, sid_hbm, w_hbm, spikes_hbm,
                       basis_hbm, out_hbm, spikes_v, basis_v, bsum_v,
                       acc_v, pre_v, post_v, sid_v, w_v):
    cid = lax.axis_index("c")
    sub = lax.axis_index("s")
    wid = cid * 16 + sub
    base = wid * _SYN_W

    pltpu.sync_copy(spikes_hbm, spikes_v)
    pltpu.sync_copy(basis_hbm, basis_v)

    lane = jnp.arange(16, dtype=jnp.int32)
    for i in range(_T // 16):
        t5 = (lane + i * 16) * _R
        s = plsc.load_gather(basis_v, [t5])
        for r in range(1, _R):
            s = s + plsc.load_gather(basis_v, [t5 + r])
        bsum_v[pl.ds(i * 16, 16)] = s

    zeros = jnp.zeros((16,), jnp.float32)

    def zero_body(i, c):
        acc_v[pl.ds(i * 16, 16)] = zeros
        return c

    lax.fori_loop(0, _ACC // 16, zero_body, 0)

    def chunk_body(c, carry):
        off = base + c * _CHUNK
        pltpu.sync_copy(pre_hbm.at[pl.ds(off, _CHUNK)], pre_v)
        pltpu.sync_copy(post_hbm.at[pl.ds(off, _CHUNK)], post_v)
        pltpu.sync_copy(sid_hbm.at[pl.ds(off, _CHUNK)], sid_v)
        pltpu.sync_copy(w_hbm.at[pl.ds(off, _CHUNK)], w_v)

        def body(i, cc):
            p = pre_v[pl.ds(i * 16, 16)]
            q = post_v[pl.ds(i * 16, 16)]
            t = sid_v[pl.ds(i * 16, 16)]
            w = w_v[pl.ds(i * 16, 16)]
            z = plsc.load_gather(spikes_v, [p])
            b = plsc.load_gather(bsum_v, [t])
            vv = z * w * b
            plsc.addupdate_scatter(acc_v, [q], vv, mask=z > 0.0)
            return cc

        lax.fori_loop(0, _VREGS, body, 0)
        return carry

    lax.fori_loop(0, _NCHUNK, chunk_body, 0)
    pltpu.sync_copy(acc_v, out_hbm.at[wid])


def _membrane_body(part_ref, v_ref, sp_ref, dec_ref, cf_ref, vth_ref,
                   nrm_ref, out_ref):
    rec = jnp.sum(part_ref[...], axis=0)
    v_reset = v_ref[...] * (1.0 - sp_ref[...])
    new_v = dec_ref[...] * v_reset + cf_ref[...] * rec
    v_scaled = (new_v - vth_ref[...]) / nrm_ref[...]
    out_ref[...] = (v_scaled > 0.0).astype(jnp.float32)


_membrane = pl.pallas_call(
    _membrane_body,
    out_shape=jax.ShapeDtypeStruct((_ACC // 128, 128), jnp.float32),
)


def kernel(spikes, v, weights, syn_ids, indices, basis, decay,
           current_factor, v_th, normalizer):
    spikes_flat = spikes.reshape(_N)
    pre = indices[:, 1]
    post = indices[:, 0]
    basis_flat = basis.reshape(_T * _R)

    partial = _sc_synapse_kernel(pre, post, syn_ids, weights, spikes_flat,
                                 basis_flat)

    def pad2d(x, fill=0.0):
        return jnp.pad(x, (0, _ACC - _N), constant_values=fill).reshape(
            _ACC // 128, 128)

    z = _membrane(partial.reshape(_NW, _ACC // 128, 128),
                  pad2d(v.reshape(_N)), pad2d(spikes_flat), pad2d(decay),
                  pad2d(current_factor), pad2d(v_th), pad2d(normalizer, 1.0))
    return z.reshape(_ACC)[:_N].reshape(1, _N)
